# baseline (speedup 1.0000x reference)
import functools

import jax
import jax.numpy as jnp
from jax import lax
from jax.experimental import pallas as pl
from jax.experimental.pallas import tpu as pltpu
from jax.experimental.pallas import tpu_sc as plsc

N_NAT = 30
N_MOV = 30
N_CENT = 9
B = 16384
M = 5
D_NAT = 64
D_MOV = 64
D_CENT = 32

NC = 2
NS = 16
L = 16
NW = NC * NS
BPW = B // NW
NV = BPW // L

ROW = 80
OFF_MOV = 32
OFF_CENT = 64
HIST_WORDS = L * ROW


def _sc_body(w_hbm, nat_hbm, mov_hbm, cent_hbm, ntab_hbm, mtab_hbm, ctab_hbm,
             part_hbm, wsum_hbm,
             w_v, nat_v, mov_v, cent_v, ntab_v, mtab_v, ctab_v,
             hist_v, svec_v, out_v, wsum_v, sem):
    wid = lax.axis_index("s") * NC + lax.axis_index("c")
    base = wid * BPW

    pltpu.sync_copy(w_hbm.at[pl.ds(base, BPW)], w_v)
    pltpu.sync_copy(nat_hbm.at[pl.ds(base, BPW)], nat_v)
    pltpu.sync_copy(mov_hbm.at[pl.ds(base * M, BPW * M)], mov_v)
    pltpu.sync_copy(cent_hbm.at[pl.ds(base, BPW)], cent_v)
    pltpu.sync_copy(ntab_hbm, ntab_v)
    pltpu.sync_copy(mtab_hbm, mtab_v)
    pltpu.sync_copy(ctab_hbm, ctab_v)

    zero = jnp.zeros((L,), jnp.float32)
    for i in range(HIST_WORDS // L):
        hist_v[pl.ds(i * L, L)] = zero

    lane = lax.iota(jnp.int32, L)
    rowbase = lane * ROW
    iota5 = lane * M
    wsum = zero
    for i in range(NV):
        o = i * L
        w = w_v[pl.ds(o, L)]
        wsum = wsum + w
        wm = w * jnp.float32(1.0 / M)
        ni = nat_v[pl.ds(o, L)]
        plsc.addupdate_scatter(hist_v, [rowbase + ni], w)
        for j in range(M):
            mj = plsc.load_gather(mov_v, [iota5 + (o * M + j)])
            plsc.addupdate_scatter(hist_v, [rowbase + (OFF_MOV + mj)], wm)
        ci = cent_v[pl.ds(o, L)]
        plsc.addupdate_scatter(hist_v, [rowbase + (OFF_CENT + ci)], w)

    for b in range(ROW // L):
        acc = hist_v[pl.ds(b * L, L)]
        for lane in range(1, L):
            acc = acc + hist_v[pl.ds(lane * ROW + b * L, L)]
        svec_v[pl.ds(b * L, L)] = acc

    accs = [jnp.zeros((L,), jnp.float32) for _ in range(10)]
    for k in range(N_NAT):
        sk = plsc.load_gather(svec_v, [jnp.full((L,), k, jnp.int32)])
        for cb in range(D_NAT // L):
            accs[cb] = accs[cb] + sk * ntab_v[pl.ds(k * D_NAT + cb * L, L)]
    for k in range(N_MOV):
        sk = plsc.load_gather(svec_v, [jnp.full((L,), OFF_MOV + k, jnp.int32)])
        for cb in range(D_MOV // L):
            accs[4 + cb] = accs[4 + cb] + sk * mtab_v[pl.ds(k * D_MOV + cb * L, L)]
    for k in range(N_CENT):
        sk = plsc.load_gather(svec_v, [jnp.full((L,), OFF_CENT + k, jnp.int32)])
        for cb in range(D_CENT // L):
            accs[8 + cb] = accs[8 + cb] + sk * ctab_v[pl.ds(k * D_CENT + cb * L, L)]

    for cb in range(10):
        out_v[pl.ds(cb * L, L)] = accs[cb]
    wsum_v[pl.ds(0, L)] = wsum
    pltpu.sync_copy(out_v, part_hbm.at[wid])
    pltpu.sync_copy(wsum_v, wsum_hbm.at[wid])


@jax.jit
def _run(weights, nat_idx, mov_flat, cent_idx, ntab, mtab, ctab):
    mesh = plsc.VectorSubcoreMesh(core_axis_name="c", subcore_axis_name="s",
                                  num_cores=NC, num_subcores=NS)
    part, wsum = pl.kernel(
        _sc_body,
        out_type=(jax.ShapeDtypeStruct((NW, 160), jnp.float32),
                  jax.ShapeDtypeStruct((NW, L), jnp.float32)),
        mesh=mesh,
        compiler_params=pltpu.CompilerParams(needs_layout_passes=False),
        scratch_types=[
            pltpu.VMEM((BPW,), jnp.float32),
            pltpu.VMEM((BPW,), jnp.int32),
            pltpu.VMEM((M * BPW,), jnp.int32),
            pltpu.VMEM((BPW,), jnp.int32),
            pltpu.VMEM((N_NAT * D_NAT,), jnp.float32),
            pltpu.VMEM((N_MOV * D_MOV,), jnp.float32),
            pltpu.VMEM((N_CENT * D_CENT,), jnp.float32),
            pltpu.VMEM((HIST_WORDS,), jnp.float32),
            pltpu.VMEM((ROW,), jnp.float32),
            pltpu.VMEM((160,), jnp.float32),
            pltpu.VMEM((L,), jnp.float32),
            pltpu.SemaphoreType.DMA,
        ],
    )(weights, nat_idx, mov_flat, cent_idx, ntab, mtab, ctab)
    return part.sum(axis=0) / wsum.sum()


def kernel(weights, nat_table, mov_table, cent_table, nat_idx, mov_idx, cent_idx):
    nat_i = nat_idx.astype(jnp.int32)
    mov_flat = mov_idx.astype(jnp.int32).reshape(-1)
    cent_i = cent_idx.astype(jnp.int32)
    return _run(weights, nat_i, mov_flat, cent_i,
                nat_table.reshape(-1), mov_table.reshape(-1),
                cent_table.reshape(-1))


# speedup vs baseline: 1.3936x; 1.3936x over previous
import functools

import jax
import jax.numpy as jnp
from jax import lax
from jax.experimental import pallas as pl
from jax.experimental.pallas import tpu as pltpu
from jax.experimental.pallas import tpu_sc as plsc

N_NAT = 30
N_MOV = 30
N_CENT = 9
B = 16384
M = 5
D_NAT = 64
D_MOV = 64
D_CENT = 32

NC = 2
NS = 16
L = 16
NW = NC * NS
BPW = B // NW
NV = BPW // L

ROW = 80
OFF_MOV = 32
OFF_CENT = 64
HIST_WORDS = L * ROW


def _sc_body(w_hbm, nat_hbm, mov_hbm, cent_hbm, ntab_hbm, mtab_hbm, ctab_hbm,
             part_hbm, wsum_hbm,
             w_v, nat_v, mov_v, cent_v, ntab_v, mtab_v, ctab_v,
             hist_v, svec_v, out_v, wsum_v, sem):
    wid = lax.axis_index("s") * NC + lax.axis_index("c")
    base = wid * BPW

    copies = [
        pltpu.async_copy(w_hbm.at[pl.ds(base, BPW)], w_v, sem),
        pltpu.async_copy(nat_hbm.at[pl.ds(base, BPW)], nat_v, sem),
        pltpu.async_copy(cent_hbm.at[pl.ds(base, BPW)], cent_v, sem),
        pltpu.async_copy(ntab_hbm, ntab_v, sem),
        pltpu.async_copy(mtab_hbm, mtab_v, sem),
        pltpu.async_copy(ctab_hbm, ctab_v, sem),
    ]
    copies += [
        pltpu.async_copy(mov_hbm.at[pl.ds(j * B + base, BPW)],
                         mov_v.at[pl.ds(j * BPW, BPW)], sem)
        for j in range(M)
    ]

    zero = jnp.zeros((L,), jnp.float32)
    for i in range(HIST_WORDS // L):
        hist_v[pl.ds(i * L, L)] = zero

    for c in copies:
        c.wait()

    lane = lax.iota(jnp.int32, L)
    rowbase = lane * ROW
    wsum = zero
    for i in range(NV):
        o = i * L
        w = w_v[pl.ds(o, L)]
        wsum = wsum + w
        wm = w * jnp.float32(1.0 / M)
        ni = nat_v[pl.ds(o, L)]
        plsc.addupdate_scatter(hist_v, [rowbase + ni], w)
        for j in range(M):
            mj = mov_v[pl.ds(j * BPW + o, L)]
            plsc.addupdate_scatter(hist_v, [rowbase + (OFF_MOV + mj)], wm)
        ci = cent_v[pl.ds(o, L)]
        plsc.addupdate_scatter(hist_v, [rowbase + (OFF_CENT + ci)], w)

    for b in range(ROW // L):
        acc = hist_v[pl.ds(b * L, L)]
        for lane in range(1, L):
            acc = acc + hist_v[pl.ds(lane * ROW + b * L, L)]
        svec_v[pl.ds(b * L, L)] = acc

    accs = [jnp.zeros((L,), jnp.float32) for _ in range(10)]
    for k in range(N_NAT):
        sk = plsc.load_gather(svec_v, [jnp.full((L,), k, jnp.int32)])
        for cb in range(D_NAT // L):
            accs[cb] = accs[cb] + sk * ntab_v[pl.ds(k * D_NAT + cb * L, L)]
    for k in range(N_MOV):
        sk = plsc.load_gather(svec_v, [jnp.full((L,), OFF_MOV + k, jnp.int32)])
        for cb in range(D_MOV // L):
            accs[4 + cb] = accs[4 + cb] + sk * mtab_v[pl.ds(k * D_MOV + cb * L, L)]
    for k in range(N_CENT):
        sk = plsc.load_gather(svec_v, [jnp.full((L,), OFF_CENT + k, jnp.int32)])
        for cb in range(D_CENT // L):
            accs[8 + cb] = accs[8 + cb] + sk * ctab_v[pl.ds(k * D_CENT + cb * L, L)]

    for cb in range(10):
        out_v[pl.ds(cb * L, L)] = accs[cb]
    wsum_v[pl.ds(0, L)] = wsum
    pltpu.sync_copy(out_v, part_hbm.at[wid])
    pltpu.sync_copy(wsum_v, wsum_hbm.at[wid])


@jax.jit
def _run(weights, nat_idx, mov_flat, cent_idx, ntab, mtab, ctab):
    mesh = plsc.VectorSubcoreMesh(core_axis_name="c", subcore_axis_name="s",
                                  num_cores=NC, num_subcores=NS)
    part, wsum = pl.kernel(
        _sc_body,
        out_type=(jax.ShapeDtypeStruct((NW, 160), jnp.float32),
                  jax.ShapeDtypeStruct((NW, L), jnp.float32)),
        mesh=mesh,
        compiler_params=pltpu.CompilerParams(needs_layout_passes=False),
        scratch_types=[
            pltpu.VMEM((BPW,), jnp.float32),
            pltpu.VMEM((BPW,), jnp.int32),
            pltpu.VMEM((M * BPW,), jnp.int32),
            pltpu.VMEM((BPW,), jnp.int32),
            pltpu.VMEM((N_NAT * D_NAT,), jnp.float32),
            pltpu.VMEM((N_MOV * D_MOV,), jnp.float32),
            pltpu.VMEM((N_CENT * D_CENT,), jnp.float32),
            pltpu.VMEM((HIST_WORDS,), jnp.float32),
            pltpu.VMEM((ROW,), jnp.float32),
            pltpu.VMEM((160,), jnp.float32),
            pltpu.VMEM((L,), jnp.float32),
            pltpu.SemaphoreType.DMA,
        ],
    )(weights, nat_idx, mov_flat, cent_idx, ntab, mtab, ctab)
    return part.sum(axis=0) / wsum.sum()


def kernel(weights, nat_table, mov_table, cent_table, nat_idx, mov_idx, cent_idx):
    nat_i = nat_idx.astype(jnp.int32)
    mov_flat = mov_idx.astype(jnp.int32).T.reshape(-1)
    cent_i = cent_idx.astype(jnp.int32)
    return _run(weights, nat_i, mov_flat, cent_i,
                nat_table.reshape(-1), mov_table.reshape(-1),
                cent_table.reshape(-1))


# speedup vs baseline: 1.4620x; 1.0491x over previous
import jax
import jax.numpy as jnp
from jax import lax
from jax.experimental import pallas as pl
from jax.experimental.pallas import tpu as pltpu
from jax.experimental.pallas import tpu_sc as plsc

N_NAT = 30
N_MOV = 30
N_CENT = 9
B = 16384
M = 5
D_NAT = 64
D_MOV = 64
D_CENT = 32

NC = 2
NS = 16
L = 16
NW = NC * NS
BPW = B // NW
NV = BPW // L

REC = BPW * (M + 3)
O_NAT = BPW
O_MOV = 2 * BPW
O_CENT = (M + 2) * BPW

T_NAT = 0
T_MOV = N_NAT * D_NAT
T_CENT = 2 * N_NAT * D_NAT
T_WORDS = 2 * N_NAT * D_NAT + N_CENT * D_CENT

ROW = 80
OFF_MOV = 32
OFF_CENT = 64
HIST_WORDS = L * ROW


def _sc_body(data_hbm, tab_hbm, part_hbm, wsum_hbm,
             data_v, tab_v, hist_v, svec_v, out_v, wsum_v, sem):
    wid = lax.axis_index("s") * NC + lax.axis_index("c")

    c0 = pltpu.async_copy(data_hbm.at[pl.ds(wid * REC, REC)], data_v, sem)
    c1 = pltpu.async_copy(tab_hbm, tab_v, sem)

    zero = jnp.zeros((L,), jnp.float32)
    for i in range(HIST_WORDS // L):
        hist_v[pl.ds(i * L, L)] = zero

    c0.wait()
    c1.wait()

    lane = lax.iota(jnp.int32, L)
    rowbase = lane * ROW
    wsum = zero
    for i in range(NV):
        o = i * L
        w = plsc.bitcast(data_v[pl.ds(o, L)], jnp.float32)
        wsum = wsum + w
        wm = w * jnp.float32(1.0 / M)
        ni = data_v[pl.ds(O_NAT + o, L)]
        plsc.addupdate_scatter(hist_v, [rowbase + ni], w)
        for j in range(M):
            mj = data_v[pl.ds(O_MOV + j * BPW + o, L)]
            plsc.addupdate_scatter(hist_v, [rowbase + (OFF_MOV + mj)], wm)
        ci = data_v[pl.ds(O_CENT + o, L)]
        plsc.addupdate_scatter(hist_v, [rowbase + (OFF_CENT + ci)], w)

    for b in range(ROW // L):
        acc = hist_v[pl.ds(b * L, L)]
        for ln in range(1, L):
            acc = acc + hist_v[pl.ds(ln * ROW + b * L, L)]
        svec_v[pl.ds(b * L, L)] = acc

    accs = [jnp.zeros((L,), jnp.float32) for _ in range(10)]
    for k in range(N_NAT):
        sk = plsc.load_gather(svec_v, [jnp.full((L,), k, jnp.int32)])
        for cb in range(D_NAT // L):
            accs[cb] = accs[cb] + sk * tab_v[pl.ds(T_NAT + k * D_NAT + cb * L, L)]
    for k in range(N_MOV):
        sk = plsc.load_gather(svec_v, [jnp.full((L,), OFF_MOV + k, jnp.int32)])
        for cb in range(D_MOV // L):
            accs[4 + cb] = accs[4 + cb] + sk * tab_v[pl.ds(T_MOV + k * D_MOV + cb * L, L)]
    for k in range(N_CENT):
        sk = plsc.load_gather(svec_v, [jnp.full((L,), OFF_CENT + k, jnp.int32)])
        for cb in range(D_CENT // L):
            accs[8 + cb] = accs[8 + cb] + sk * tab_v[pl.ds(T_CENT + k * D_CENT + cb * L, L)]

    for cb in range(10):
        out_v[pl.ds(cb * L, L)] = accs[cb]
    wsum_v[pl.ds(0, L)] = wsum
    pltpu.sync_copy(out_v, part_hbm.at[wid])
    pltpu.sync_copy(wsum_v, wsum_hbm.at[wid])


@jax.jit
def _run(data, tab):
    mesh = plsc.VectorSubcoreMesh(core_axis_name="c", subcore_axis_name="s",
                                  num_cores=NC, num_subcores=NS)
    part, wsum = pl.kernel(
        _sc_body,
        out_type=(jax.ShapeDtypeStruct((NW, 160), jnp.float32),
                  jax.ShapeDtypeStruct((NW, L), jnp.float32)),
        mesh=mesh,
        compiler_params=pltpu.CompilerParams(needs_layout_passes=False),
        scratch_types=[
            pltpu.VMEM((REC,), jnp.int32),
            pltpu.VMEM((T_WORDS,), jnp.float32),
            pltpu.VMEM((HIST_WORDS,), jnp.float32),
            pltpu.VMEM((ROW,), jnp.float32),
            pltpu.VMEM((160,), jnp.float32),
            pltpu.VMEM((L,), jnp.float32),
            pltpu.SemaphoreType.DMA,
        ],
    )(data, tab)
    return part.sum(axis=0) / wsum.sum()


def kernel(weights, nat_table, mov_table, cent_table, nat_idx, mov_idx, cent_idx):
    w_i = jax.lax.bitcast_convert_type(weights, jnp.int32).reshape(NW, BPW)
    nat_i = nat_idx.astype(jnp.int32).reshape(NW, BPW)
    mov_i = (mov_idx.astype(jnp.int32).T
             .reshape(M, NW, BPW).transpose(1, 0, 2).reshape(NW, M * BPW))
    cent_i = cent_idx.astype(jnp.int32).reshape(NW, BPW)
    data = jnp.concatenate([w_i, nat_i, mov_i, cent_i], axis=1).reshape(-1)
    tab = jnp.concatenate([nat_table.reshape(-1), mov_table.reshape(-1),
                           cent_table.reshape(-1)])
    return _run(data, tab)


# speedup vs baseline: 1.5394x; 1.0529x over previous
import jax
import jax.numpy as jnp
from jax import lax
from jax.experimental import pallas as pl
from jax.experimental.pallas import tpu as pltpu
from jax.experimental.pallas import tpu_sc as plsc

N_NAT = 30
N_MOV = 30
N_CENT = 9
B = 16384
M = 5
D_NAT = 64
D_MOV = 64
D_CENT = 32

NC = 2
NS = 16
L = 16
NW = NC * NS
BPW = B // NW
NV = BPW // L

REC = BPW * (M + 3)
O_NAT = BPW
O_MOV = 2 * BPW
O_CENT = (M + 2) * BPW

T_NAT = 0
T_MOV = N_NAT * D_NAT
T_CENT = 2 * N_NAT * D_NAT
T_WORDS = 2 * N_NAT * D_NAT + N_CENT * D_CENT

ROW = 80
OFF_MOV = 32
OFF_CENT = 64
HIST_WORDS = L * ROW


def _sc_body(data_hbm, tab_hbm, part_hbm, wsum_hbm,
             data_v, tab_v, hist_v, svec_v, out_v, wsum_v, sem):
    wid = lax.axis_index("s") * NC + lax.axis_index("c")

    c0 = pltpu.async_copy(data_hbm.at[pl.ds(wid * REC, REC)], data_v, sem)
    c1 = pltpu.async_copy(tab_hbm, tab_v, sem)

    zero = jnp.zeros((L,), jnp.float32)

    def zero_step(i, _):
        hist_v[pl.ds(i * L, L)] = zero
        return 0
    lax.fori_loop(0, HIST_WORDS // L, zero_step, 0)

    c0.wait()
    c1.wait()

    lane = lax.iota(jnp.int32, L)
    rowbase = lane * ROW

    def hist_step(i, wsum):
        o = i * L
        w = plsc.bitcast(data_v[pl.ds(o, L)], jnp.float32)
        wm = w * jnp.float32(1.0 / M)
        ni = data_v[pl.ds(O_NAT + o, L)]
        plsc.addupdate_scatter(hist_v, [rowbase + ni], w)
        for j in range(M):
            mj = data_v[pl.ds(O_MOV + j * BPW + o, L)]
            plsc.addupdate_scatter(hist_v, [rowbase + (OFF_MOV + mj)], wm)
        ci = data_v[pl.ds(O_CENT + o, L)]
        plsc.addupdate_scatter(hist_v, [rowbase + (OFF_CENT + ci)], w)
        return wsum + w

    wsum = lax.fori_loop(0, NV, hist_step, zero)

    for b in range(ROW // L):
        def red_step(ln, acc, b=b):
            return acc + hist_v[pl.ds(ln * ROW + b * L, L)]
        svec_v[pl.ds(b * L, L)] = lax.fori_loop(0, L, red_step, zero)

    def mv_nat(k, accs):
        sk = plsc.load_gather(svec_v, [jnp.full((L,), 1, jnp.int32) * k])
        return tuple(a + sk * tab_v[pl.ds(T_NAT + k * D_NAT + cb * L, L)]
                     for cb, a in enumerate(accs))
    def mv_mov(k, accs):
        sk = plsc.load_gather(svec_v, [jnp.full((L,), 1, jnp.int32) * (OFF_MOV + k)])
        return tuple(a + sk * tab_v[pl.ds(T_MOV + k * D_MOV + cb * L, L)]
                     for cb, a in enumerate(accs))
    def mv_cent(k, accs):
        sk = plsc.load_gather(svec_v, [jnp.full((L,), 1, jnp.int32) * (OFF_CENT + k)])
        return tuple(a + sk * tab_v[pl.ds(T_CENT + k * D_CENT + cb * L, L)]
                     for cb, a in enumerate(accs))

    acc_nat = lax.fori_loop(0, N_NAT, mv_nat, (zero,) * 4)
    acc_mov = lax.fori_loop(0, N_MOV, mv_mov, (zero,) * 4)
    acc_cent = lax.fori_loop(0, N_CENT, mv_cent, (zero,) * 2)

    for cb, a in enumerate(acc_nat + acc_mov + acc_cent):
        out_v[pl.ds(cb * L, L)] = a
    wsum_v[pl.ds(0, L)] = wsum
    pltpu.sync_copy(out_v, part_hbm.at[wid])
    pltpu.sync_copy(wsum_v, wsum_hbm.at[wid])


@jax.jit
def _run(data, tab):
    mesh = plsc.VectorSubcoreMesh(core_axis_name="c", subcore_axis_name="s",
                                  num_cores=NC, num_subcores=NS)
    part, wsum = pl.kernel(
        _sc_body,
        out_type=(jax.ShapeDtypeStruct((NW, 160), jnp.float32),
                  jax.ShapeDtypeStruct((NW, L), jnp.float32)),
        mesh=mesh,
        compiler_params=pltpu.CompilerParams(needs_layout_passes=False),
        scratch_types=[
            pltpu.VMEM((REC,), jnp.int32),
            pltpu.VMEM((T_WORDS,), jnp.float32),
            pltpu.VMEM((HIST_WORDS,), jnp.float32),
            pltpu.VMEM((ROW,), jnp.float32),
            pltpu.VMEM((160,), jnp.float32),
            pltpu.VMEM((L,), jnp.float32),
            pltpu.SemaphoreType.DMA,
        ],
    )(data, tab)
    return part.sum(axis=0) / wsum.sum()


def kernel(weights, nat_table, mov_table, cent_table, nat_idx, mov_idx, cent_idx):
    w_i = jax.lax.bitcast_convert_type(weights, jnp.int32).reshape(NW, BPW)
    nat_i = nat_idx.astype(jnp.int32).reshape(NW, BPW)
    mov_i = (mov_idx.astype(jnp.int32).T
             .reshape(M, NW, BPW).transpose(1, 0, 2).reshape(NW, M * BPW))
    cent_i = cent_idx.astype(jnp.int32).reshape(NW, BPW)
    data = jnp.concatenate([w_i, nat_i, mov_i, cent_i], axis=1).reshape(-1)
    tab = jnp.concatenate([nat_table.reshape(-1), mov_table.reshape(-1),
                           cent_table.reshape(-1)])
    return _run(data, tab)


# speedup vs baseline: 1.5883x; 1.0318x over previous
import jax
import jax.numpy as jnp
from jax import lax
from jax.experimental import pallas as pl
from jax.experimental.pallas import tpu as pltpu
from jax.experimental.pallas import tpu_sc as plsc

N_NAT = 30
N_MOV = 30
N_CENT = 9
B = 16384
M = 5
D_NAT = 64
D_MOV = 64
D_CENT = 32

NC = 2
NS = 16
L = 16
NW = NC * NS
BPW = B // NW
NV = BPW // L

REC = BPW * (M + 3)
O_NAT = BPW
O_MOV = 2 * BPW
O_CENT = (M + 2) * BPW

T_NAT = 0
T_MOV = N_NAT * D_NAT
T_CENT = 2 * N_NAT * D_NAT
T_WORDS = 2 * N_NAT * D_NAT + N_CENT * D_CENT

ROW = 80
OFF_MOV = 32
OFF_CENT = 64
HIST_WORDS = L * ROW


def _sc_body(data_hbm, tab_hbm, part_hbm, wsum_hbm,
             data_v, tab_v, hist_v, svec_v, out_v, wsum_v, sem):
    wid = lax.axis_index("s") * NC + lax.axis_index("c")

    c0 = pltpu.async_copy(data_hbm.at[pl.ds(wid * REC, REC)], data_v, sem)
    c1 = pltpu.async_copy(tab_hbm, tab_v, sem)

    zero = jnp.zeros((L,), jnp.float32)

    def zero_step(i, _):
        hist_v[pl.ds(i * L, L)] = zero
        return 0
    lax.fori_loop(0, HIST_WORDS // L, zero_step, 0)

    c0.wait()
    c1.wait()

    lane = lax.iota(jnp.int32, L)
    rowbase = lane * ROW

    @plsc.parallel_loop(0, NV, unroll=4, carry=zero)
    def hist_step(i, wsum):
        o = i * L
        w = plsc.bitcast(data_v[pl.ds(o, L)], jnp.float32)
        wm = w * jnp.float32(1.0 / M)
        ni = data_v[pl.ds(O_NAT + o, L)]
        plsc.addupdate_scatter(hist_v, [rowbase + ni], w)
        for j in range(M):
            mj = data_v[pl.ds(O_MOV + j * BPW + o, L)]
            plsc.addupdate_scatter(hist_v, [rowbase + (OFF_MOV + mj)], wm)
        ci = data_v[pl.ds(O_CENT + o, L)]
        plsc.addupdate_scatter(hist_v, [rowbase + (OFF_CENT + ci)], w)
        return wsum + w

    wsum = hist_step

    for b in range(ROW // L):
        def red_step(ln, acc, b=b):
            return acc + hist_v[pl.ds(ln * ROW + b * L, L)]
        svec_v[pl.ds(b * L, L)] = lax.fori_loop(0, L, red_step, zero)

    def mv_nat(k, accs):
        sk = plsc.load_gather(svec_v, [jnp.full((L,), 1, jnp.int32) * k])
        return tuple(a + sk * tab_v[pl.ds(T_NAT + k * D_NAT + cb * L, L)]
                     for cb, a in enumerate(accs))
    def mv_mov(k, accs):
        sk = plsc.load_gather(svec_v, [jnp.full((L,), 1, jnp.int32) * (OFF_MOV + k)])
        return tuple(a + sk * tab_v[pl.ds(T_MOV + k * D_MOV + cb * L, L)]
                     for cb, a in enumerate(accs))
    def mv_cent(k, accs):
        sk = plsc.load_gather(svec_v, [jnp.full((L,), 1, jnp.int32) * (OFF_CENT + k)])
        return tuple(a + sk * tab_v[pl.ds(T_CENT + k * D_CENT + cb * L, L)]
                     for cb, a in enumerate(accs))

    acc_nat = lax.fori_loop(0, N_NAT, mv_nat, (zero,) * 4)
    acc_mov = lax.fori_loop(0, N_MOV, mv_mov, (zero,) * 4)
    acc_cent = lax.fori_loop(0, N_CENT, mv_cent, (zero,) * 2)

    for cb, a in enumerate(acc_nat + acc_mov + acc_cent):
        out_v[pl.ds(cb * L, L)] = a
    wsum_v[pl.ds(0, L)] = wsum
    pltpu.sync_copy(out_v, part_hbm.at[wid])
    pltpu.sync_copy(wsum_v, wsum_hbm.at[wid])


@jax.jit
def _run(data, tab):
    mesh = plsc.VectorSubcoreMesh(core_axis_name="c", subcore_axis_name="s",
                                  num_cores=NC, num_subcores=NS)
    part, wsum = pl.kernel(
        _sc_body,
        out_type=(jax.ShapeDtypeStruct((NW, 160), jnp.float32),
                  jax.ShapeDtypeStruct((NW, L), jnp.float32)),
        mesh=mesh,
        compiler_params=pltpu.CompilerParams(needs_layout_passes=False),
        scratch_types=[
            pltpu.VMEM((REC,), jnp.int32),
            pltpu.VMEM((T_WORDS,), jnp.float32),
            pltpu.VMEM((HIST_WORDS,), jnp.float32),
            pltpu.VMEM((ROW,), jnp.float32),
            pltpu.VMEM((160,), jnp.float32),
            pltpu.VMEM((L,), jnp.float32),
            pltpu.SemaphoreType.DMA,
        ],
    )(data, tab)
    return part.sum(axis=0) / wsum.sum()


def kernel(weights, nat_table, mov_table, cent_table, nat_idx, mov_idx, cent_idx):
    w_i = jax.lax.bitcast_convert_type(weights, jnp.int32).reshape(NW, BPW)
    nat_i = nat_idx.astype(jnp.int32).reshape(NW, BPW)
    mov_i = (mov_idx.astype(jnp.int32).T
             .reshape(M, NW, BPW).transpose(1, 0, 2).reshape(NW, M * BPW))
    cent_i = cent_idx.astype(jnp.int32).reshape(NW, BPW)
    data = jnp.concatenate([w_i, nat_i, mov_i, cent_i], axis=1).reshape(-1)
    tab = jnp.concatenate([nat_table.reshape(-1), mov_table.reshape(-1),
                           cent_table.reshape(-1)])
    return _run(data, tab)


# speedup vs baseline: 1.6721x; 1.0528x over previous
import jax
import jax.numpy as jnp
from jax import lax
from jax.experimental import pallas as pl
from jax.experimental.pallas import tpu as pltpu
from jax.experimental.pallas import tpu_sc as plsc

N_NAT = 30
N_MOV = 30
N_CENT = 9
B = 16384
M = 5
D_NAT = 64
D_MOV = 64
D_CENT = 32

NC = 1
NS = 16
L = 16
NW = NC * NS
BPW = B // NW
NV = BPW // L

REC = BPW * (M + 3)
O_NAT = BPW
O_MOV = 2 * BPW
O_CENT = (M + 2) * BPW

T_NAT = 0
T_MOV = N_NAT * D_NAT
T_CENT = 2 * N_NAT * D_NAT
T_WORDS = 2 * N_NAT * D_NAT + N_CENT * D_CENT

ROW = 80
OFF_MOV = 32
OFF_CENT = 64
HIST_WORDS = L * ROW


def _sc_body(data_hbm, tab_hbm, part_hbm, wsum_hbm,
             data_v, tab_v, hist_v, svec_v, out_v, wsum_v, sem):
    wid = lax.axis_index("s") * NC + lax.axis_index("c")

    c0 = pltpu.async_copy(data_hbm.at[pl.ds(wid * REC, REC)], data_v, sem)
    c1 = pltpu.async_copy(tab_hbm, tab_v, sem)

    zero = jnp.zeros((L,), jnp.float32)

    def zero_step(i, _):
        hist_v[pl.ds(i * L, L)] = zero
        return 0
    lax.fori_loop(0, HIST_WORDS // L, zero_step, 0)

    c0.wait()
    c1.wait()

    lane = lax.iota(jnp.int32, L)
    rowbase = lane * ROW

    @plsc.parallel_loop(0, NV, unroll=4, carry=zero)
    def hist_step(i, wsum):
        o = i * L
        w = plsc.bitcast(data_v[pl.ds(o, L)], jnp.float32)
        wm = w * jnp.float32(1.0 / M)
        ni = data_v[pl.ds(O_NAT + o, L)]
        plsc.addupdate_scatter(hist_v, [rowbase + ni], w)
        for j in range(M):
            mj = data_v[pl.ds(O_MOV + j * BPW + o, L)]
            plsc.addupdate_scatter(hist_v, [rowbase + (OFF_MOV + mj)], wm)
        ci = data_v[pl.ds(O_CENT + o, L)]
        plsc.addupdate_scatter(hist_v, [rowbase + (OFF_CENT + ci)], w)
        return wsum + w

    wsum = hist_step

    for b in range(ROW // L):
        def red_step(ln, acc, b=b):
            return acc + hist_v[pl.ds(ln * ROW + b * L, L)]
        svec_v[pl.ds(b * L, L)] = lax.fori_loop(0, L, red_step, zero)

    def mv_nat(k, accs):
        sk = plsc.load_gather(svec_v, [jnp.full((L,), 1, jnp.int32) * k])
        return tuple(a + sk * tab_v[pl.ds(T_NAT + k * D_NAT + cb * L, L)]
                     for cb, a in enumerate(accs))
    def mv_mov(k, accs):
        sk = plsc.load_gather(svec_v, [jnp.full((L,), 1, jnp.int32) * (OFF_MOV + k)])
        return tuple(a + sk * tab_v[pl.ds(T_MOV + k * D_MOV + cb * L, L)]
                     for cb, a in enumerate(accs))
    def mv_cent(k, accs):
        sk = plsc.load_gather(svec_v, [jnp.full((L,), 1, jnp.int32) * (OFF_CENT + k)])
        return tuple(a + sk * tab_v[pl.ds(T_CENT + k * D_CENT + cb * L, L)]
                     for cb, a in enumerate(accs))

    acc_nat = lax.fori_loop(0, N_NAT, mv_nat, (zero,) * 4)
    acc_mov = lax.fori_loop(0, N_MOV, mv_mov, (zero,) * 4)
    acc_cent = lax.fori_loop(0, N_CENT, mv_cent, (zero,) * 2)

    for cb, a in enumerate(acc_nat + acc_mov + acc_cent):
        out_v[pl.ds(cb * L, L)] = a
    wsum_v[pl.ds(0, L)] = wsum
    pltpu.sync_copy(out_v, part_hbm.at[wid])
    pltpu.sync_copy(wsum_v, wsum_hbm.at[wid])


@jax.jit
def _run(data, tab):
    mesh = plsc.VectorSubcoreMesh(core_axis_name="c", subcore_axis_name="s",
                                  num_cores=NC, num_subcores=NS)
    part, wsum = pl.kernel(
        _sc_body,
        out_type=(jax.ShapeDtypeStruct((NW, 160), jnp.float32),
                  jax.ShapeDtypeStruct((NW, L), jnp.float32)),
        mesh=mesh,
        compiler_params=pltpu.CompilerParams(needs_layout_passes=False),
        scratch_types=[
            pltpu.VMEM((REC,), jnp.int32),
            pltpu.VMEM((T_WORDS,), jnp.float32),
            pltpu.VMEM((HIST_WORDS,), jnp.float32),
            pltpu.VMEM((ROW,), jnp.float32),
            pltpu.VMEM((160,), jnp.float32),
            pltpu.VMEM((L,), jnp.float32),
            pltpu.SemaphoreType.DMA,
        ],
    )(data, tab)
    return part.sum(axis=0) / wsum.sum()


def kernel(weights, nat_table, mov_table, cent_table, nat_idx, mov_idx, cent_idx):
    w_i = jax.lax.bitcast_convert_type(weights, jnp.int32).reshape(NW, BPW)
    nat_i = nat_idx.astype(jnp.int32).reshape(NW, BPW)
    mov_i = (mov_idx.astype(jnp.int32).T
             .reshape(M, NW, BPW).transpose(1, 0, 2).reshape(NW, M * BPW))
    cent_i = cent_idx.astype(jnp.int32).reshape(NW, BPW)
    data = jnp.concatenate([w_i, nat_i, mov_i, cent_i], axis=1).reshape(-1)
    tab = jnp.concatenate([nat_table.reshape(-1), mov_table.reshape(-1),
                           cent_table.reshape(-1)])
    return _run(data, tab)


# speedup vs baseline: 1.8099x; 1.0824x over previous
import jax
import jax.numpy as jnp
from jax import lax
from jax.experimental import pallas as pl
from jax.experimental.pallas import tpu as pltpu
from jax.experimental.pallas import tpu_sc as plsc

N_NAT = 30
N_MOV = 30
N_CENT = 9
B = 16384
M = 5
D_NAT = 64
D_MOV = 64
D_CENT = 32

NC = 1
NS = 16
L = 16
NW = NC * NS
BPW = B // NW
NV = BPW // L

REC = BPW * (M + 3)
O_NAT = BPW
O_MOV = 2 * BPW
O_CENT = (M + 2) * BPW

T_NAT = 0
T_MOV = N_NAT * D_NAT
T_CENT = 2 * N_NAT * D_NAT
T_WORDS = 2 * N_NAT * D_NAT + N_CENT * D_CENT

ROW = 80
OFF_MOV = 32
OFF_CENT = 64
HIST_WORDS = L * ROW


def _sc_body(data_hbm, tab_hbm, out_hbm,
             data_v, tab_v, hist_v, svec_v, out_v, shared_v, red_v, fin_v, sem):
    wid = lax.axis_index("s") * NC + lax.axis_index("c")

    c0 = pltpu.async_copy(data_hbm.at[pl.ds(wid * REC, REC)], data_v, sem)
    c1 = pltpu.async_copy(tab_hbm, tab_v, sem)

    zero = jnp.zeros((L,), jnp.float32)

    def zero_step(i, _):
        hist_v[pl.ds(i * L, L)] = zero
        return 0
    lax.fori_loop(0, HIST_WORDS // L, zero_step, 0)

    c0.wait()
    c1.wait()

    lane = lax.iota(jnp.int32, L)
    rowbase = lane * ROW

    @plsc.parallel_loop(0, NV, unroll=4, carry=zero)
    def hist_step(i, wsum):
        o = i * L
        w = plsc.bitcast(data_v[pl.ds(o, L)], jnp.float32)
        wm = w * jnp.float32(1.0 / M)
        ni = data_v[pl.ds(O_NAT + o, L)]
        plsc.addupdate_scatter(hist_v, [rowbase + ni], w)
        for j in range(M):
            mj = data_v[pl.ds(O_MOV + j * BPW + o, L)]
            plsc.addupdate_scatter(hist_v, [rowbase + (OFF_MOV + mj)], wm)
        ci = data_v[pl.ds(O_CENT + o, L)]
        plsc.addupdate_scatter(hist_v, [rowbase + (OFF_CENT + ci)], w)
        return wsum + w

    wsum = hist_step

    for b in range(ROW // L):
        def red_step(ln, acc, b=b):
            return acc + hist_v[pl.ds(ln * ROW + b * L, L)]
        svec_v[pl.ds(b * L, L)] = lax.fori_loop(0, L, red_step, zero)

    def mv_nat(k, accs):
        sk = plsc.load_gather(svec_v, [jnp.full((L,), 1, jnp.int32) * k])
        return tuple(a + sk * tab_v[pl.ds(T_NAT + k * D_NAT + cb * L, L)]
                     for cb, a in enumerate(accs))
    def mv_mov(k, accs):
        sk = plsc.load_gather(svec_v, [jnp.full((L,), 1, jnp.int32) * (OFF_MOV + k)])
        return tuple(a + sk * tab_v[pl.ds(T_MOV + k * D_MOV + cb * L, L)]
                     for cb, a in enumerate(accs))
    def mv_cent(k, accs):
        sk = plsc.load_gather(svec_v, [jnp.full((L,), 1, jnp.int32) * (OFF_CENT + k)])
        return tuple(a + sk * tab_v[pl.ds(T_CENT + k * D_CENT + cb * L, L)]
                     for cb, a in enumerate(accs))

    acc_nat = lax.fori_loop(0, N_NAT, mv_nat, (zero,) * 4)
    acc_mov = lax.fori_loop(0, N_MOV, mv_mov, (zero,) * 4)
    acc_cent = lax.fori_loop(0, N_CENT, mv_cent, (zero,) * 2)

    for cb, a in enumerate(acc_nat + acc_mov + acc_cent):
        out_v[pl.ds(cb * L, L)] = a
    out_v[pl.ds(160, L)] = wsum

    pltpu.sync_copy(out_v, shared_v.at[wid])
    plsc.subcore_barrier()

    @pl.when(wid == 0)
    def _():
        pltpu.sync_copy(shared_v, red_v)

        def tot_step(ln, accs):
            return tuple(a + red_v[ln, pl.ds(blk * L, L)]
                         for blk, a in enumerate(accs))
        totals = lax.fori_loop(0, NS, tot_step, (zero,) * 11)
        wvec = jnp.full((L,), 1.0, jnp.float32) * jnp.sum(totals[10])
        inv = jnp.full((L,), 1.0, jnp.float32) / wvec
        for blk in range(10):
            fin_v[pl.ds(blk * L, L)] = totals[blk] * inv
        pltpu.sync_copy(fin_v, out_hbm)


@jax.jit
def _run(data, tab):
    mesh = plsc.VectorSubcoreMesh(core_axis_name="c", subcore_axis_name="s",
                                  num_cores=NC, num_subcores=NS)
    out = pl.kernel(
        _sc_body,
        out_type=jax.ShapeDtypeStruct((160,), jnp.float32),
        mesh=mesh,
        compiler_params=pltpu.CompilerParams(needs_layout_passes=False),
        scratch_types=[
            pltpu.VMEM((REC,), jnp.int32),
            pltpu.VMEM((T_WORDS,), jnp.float32),
            pltpu.VMEM((HIST_WORDS,), jnp.float32),
            pltpu.VMEM((ROW,), jnp.float32),
            pltpu.VMEM((176,), jnp.float32),
            pltpu.VMEM_SHARED((NW, 176), jnp.float32),
            pltpu.VMEM((NW, 176), jnp.float32),
            pltpu.VMEM((160,), jnp.float32),
            pltpu.SemaphoreType.DMA,
        ],
    )(data, tab)
    return out


def kernel(weights, nat_table, mov_table, cent_table, nat_idx, mov_idx, cent_idx):
    w_i = jax.lax.bitcast_convert_type(weights, jnp.int32).reshape(NW, BPW)
    nat_i = nat_idx.astype(jnp.int32).reshape(NW, BPW)
    mov_i = (mov_idx.astype(jnp.int32).T
             .reshape(M, NW, BPW).transpose(1, 0, 2).reshape(NW, M * BPW))
    cent_i = cent_idx.astype(jnp.int32).reshape(NW, BPW)
    data = jnp.concatenate([w_i, nat_i, mov_i, cent_i], axis=1).reshape(-1)
    tab = jnp.concatenate([nat_table.reshape(-1), mov_table.reshape(-1),
                           cent_table.reshape(-1)])
    return _run(data, tab)
